# baseline (speedup 1.0000x reference)
import jax
import jax.numpy as jnp
from jax.experimental import pallas as pl
from jax.experimental.pallas import tpu as pltpu

_BLK = 10000


def _fused_block(obs_ref, x_ref, e_ref, pw1, pb1, pw2, pb2, nw1, nb1, nw2, nb2,
                 out_ref, eout_ref):
    x = x_ref[...]
    obs = obs_ref[...]
    f32 = jnp.float32
    bf = jnp.bfloat16
    xb = x.astype(bf)
    hp = jnp.maximum(jax.lax.dot(xb, pw1[...], preferred_element_type=f32) + pb1[...], 0.0)
    yp = jnp.maximum(jax.lax.dot(hp.astype(bf), pw2[...], preferred_element_type=f32) + pb2[...], 0.0)
    hn = jnp.maximum(jax.lax.dot(xb, nw1[...], preferred_element_type=f32) + nb1[...], 0.0)
    yn = jnp.maximum(jax.lax.dot(hn.astype(bf), nw2[...], preferred_element_type=f32) + nb2[...], 0.0)
    out_ref[...] = jnp.where(obs == 2, yp, jnp.where(obs == 0, yn, x))
    eout_ref[...] = e_ref[...]


def kernel(node_data, observations, edge_weights, pos_W1, pos_b1, pos_W2, pos_b2,
           neg_W1, neg_b1, neg_W2, neg_b2):
    n, d = node_data.shape
    e = edge_weights.shape[0]
    blk = _BLK
    g = n // blk
    obs = observations.astype(jnp.int32).reshape(n, 1)
    full = lambda i: (0, 0)
    perm = lambda i: ((i * 3) % g, 0)
    wspec = pl.BlockSpec((d, d), full)
    bspec = pl.BlockSpec((1, d), full)
    out, eout = pl.pallas_call(
        _fused_block,
        grid=(g,),
        in_specs=[
            pl.BlockSpec((blk, 1), perm),
            pl.BlockSpec((blk, d), perm),
            pl.BlockSpec((e,), lambda i: (0,)),
            wspec, bspec, wspec, bspec,
            wspec, bspec, wspec, bspec,
        ],
        out_specs=[
            pl.BlockSpec((blk, d), perm),
            pl.BlockSpec((e,), lambda i: (0,)),
        ],
        out_shape=[
            jax.ShapeDtypeStruct((n, d), jnp.float32),
            jax.ShapeDtypeStruct((e,), jnp.float32),
        ],
        compiler_params=pltpu.CompilerParams(vmem_limit_bytes=114 * 1024 * 1024),
    )(
        obs, node_data, edge_weights,
        pos_W1.T.astype(jnp.bfloat16), pos_b1.reshape(1, d), pos_W2.T.astype(jnp.bfloat16), pos_b2.reshape(1, d),
        neg_W1.T.astype(jnp.bfloat16), neg_b1.reshape(1, d), neg_W2.T.astype(jnp.bfloat16), neg_b2.reshape(1, d),
    )
    return out, eout


# speedup vs baseline: 1.0618x; 1.0618x over previous
import jax
import jax.numpy as jnp
from jax.experimental import pallas as pl
from jax.experimental.pallas import tpu as pltpu

_BLK = 10000


def _fused_block(obs_ref, x_ref, e_ref, w1c, b1c, w2c, pb2, nb2, out_ref, eout_ref):
    f32 = jnp.float32
    bf = jnp.bfloat16
    x = x_ref[...]
    obs = obs_ref[...]
    mpos = obs == 2
    mneg = obs == 0
    d = x.shape[1]
    xb = x.astype(bf)
    h_raw = jax.lax.dot(xb, w1c[...], preferred_element_type=f32)
    b1 = b1c[...]
    hp = jnp.where(mpos, jnp.maximum(h_raw[:, :d] + b1[:, :d], 0.0), 0.0).astype(bf)
    hn = jnp.where(mneg, jnp.maximum(h_raw[:, d:] + b1[:, d:], 0.0), 0.0).astype(bf)
    h = jnp.concatenate([hp, hn], axis=1)
    y_raw = jax.lax.dot(h, w2c[...], preferred_element_type=f32)
    b2 = jnp.where(mpos, pb2[...], nb2[...])
    out_ref[...] = jnp.where(mpos | mneg, jnp.maximum(y_raw + b2, 0.0), x)
    eout_ref[...] = e_ref[...]


def kernel(node_data, observations, edge_weights, pos_W1, pos_b1, pos_W2, pos_b2,
           neg_W1, neg_b1, neg_W2, neg_b2):
    n, d = node_data.shape
    e = edge_weights.shape[0]
    blk = _BLK
    g = n // blk
    obs = observations.astype(jnp.int32).reshape(n, 1)
    bf = jnp.bfloat16
    w1c = jnp.concatenate([pos_W1.T, neg_W1.T], axis=1).astype(bf)
    b1c = jnp.concatenate([pos_b1, neg_b1]).reshape(1, 2 * d)
    w2c = jnp.concatenate([pos_W2.T, neg_W2.T], axis=0).astype(bf)
    full = lambda i: (0, 0)
    perm = lambda i: ((i * 3) % g, 0)
    out, eout = pl.pallas_call(
        _fused_block,
        grid=(g,),
        in_specs=[
            pl.BlockSpec((blk, 1), perm),
            pl.BlockSpec((blk, d), perm),
            pl.BlockSpec((e,), lambda i: (0,)),
            pl.BlockSpec((d, 2 * d), full),
            pl.BlockSpec((1, 2 * d), full),
            pl.BlockSpec((2 * d, d), full),
            pl.BlockSpec((1, d), full),
            pl.BlockSpec((1, d), full),
        ],
        out_specs=[
            pl.BlockSpec((blk, d), perm),
            pl.BlockSpec((e,), lambda i: (0,)),
        ],
        out_shape=[
            jax.ShapeDtypeStruct((n, d), jnp.float32),
            jax.ShapeDtypeStruct((e,), jnp.float32),
        ],
        compiler_params=pltpu.CompilerParams(vmem_limit_bytes=114 * 1024 * 1024),
    )(
        obs, node_data, edge_weights,
        w1c, b1c, w2c, pos_b2.reshape(1, d), neg_b2.reshape(1, d),
    )
    return out, eout


# speedup vs baseline: 1.0849x; 1.0217x over previous
import jax
import jax.numpy as jnp
from jax.experimental import pallas as pl
from jax.experimental.pallas import tpu as pltpu

_BLK = 10000


def _fused_block(obs_ref, x_ref, e_ref, w1c, b1c, w2c, pb2, nb2, out_ref, eout_ref):
    f32 = jnp.float32
    bf = jnp.bfloat16
    x = x_ref[...]
    obs = obs_ref[...]
    mpos = obs == 2
    mneg = obs == 0
    d = x.shape[1]
    xb = x.astype(bf)
    h_raw = jax.lax.dot(xb, w1c[...], preferred_element_type=f32)
    b1 = b1c[...]
    hp = jnp.where(mpos, jnp.maximum(h_raw[:, :d] + b1[:, :d], 0.0), 0.0).astype(bf)
    hn = jnp.where(mneg, jnp.maximum(h_raw[:, d:] + b1[:, d:], 0.0), 0.0).astype(bf)
    h = jnp.concatenate([hp, hn], axis=1)
    y_raw = jax.lax.dot(h, w2c[...], preferred_element_type=f32)
    b2 = jnp.where(mpos, pb2[...], nb2[...])
    out_ref[...] = jnp.where(obs == 1, x, jnp.maximum(y_raw + b2, 0.0))

    @pl.when(pl.program_id(0) == 0)
    def _copy_edges():
        eout_ref[...] = e_ref[...]


def kernel(node_data, observations, edge_weights, pos_W1, pos_b1, pos_W2, pos_b2,
           neg_W1, neg_b1, neg_W2, neg_b2):
    n, d = node_data.shape
    e = edge_weights.shape[0]
    blk = _BLK
    g = n // blk
    obs = observations.astype(jnp.int32).reshape(n, 1)
    bf = jnp.bfloat16
    w1c = jnp.concatenate([pos_W1.T, neg_W1.T], axis=1).astype(bf)
    b1c = jnp.concatenate([pos_b1, neg_b1]).reshape(1, 2 * d)
    w2c = jnp.concatenate([pos_W2.T, neg_W2.T], axis=0).astype(bf)
    full = lambda i: (0, 0)
    perm = lambda i: ((i * 3) % g, 0)
    out, eout = pl.pallas_call(
        _fused_block,
        grid=(g,),
        in_specs=[
            pl.BlockSpec((blk, 1), perm),
            pl.BlockSpec((blk, d), perm),
            pl.BlockSpec((e,), lambda i: (0,)),
            pl.BlockSpec((d, 2 * d), full),
            pl.BlockSpec((1, 2 * d), full),
            pl.BlockSpec((2 * d, d), full),
            pl.BlockSpec((1, d), full),
            pl.BlockSpec((1, d), full),
        ],
        out_specs=[
            pl.BlockSpec((blk, d), perm),
            pl.BlockSpec((e,), lambda i: (0,)),
        ],
        out_shape=[
            jax.ShapeDtypeStruct((n, d), jnp.float32),
            jax.ShapeDtypeStruct((e,), jnp.float32),
        ],
        compiler_params=pltpu.CompilerParams(vmem_limit_bytes=114 * 1024 * 1024),
    )(
        obs, node_data, edge_weights,
        w1c, b1c, w2c, pos_b2.reshape(1, d), neg_b2.reshape(1, d),
    )
    return out, eout


# speedup vs baseline: 2.1223x; 1.9561x over previous
import jax
import jax.numpy as jnp
from jax.experimental import pallas as pl
from jax.experimental.pallas import tpu as pltpu

_BLK = 10000


def _fused_block(obs_ref, x_ref, e_ref, w1c, b1c, w2c, pb2, nb2, out_ref, eout_ref):
    f32 = jnp.float32
    bf = jnp.bfloat16
    x = x_ref[...]
    blk, d0 = x.shape
    obs_row = obs_ref[...].reshape(1, blk)
    obs = jnp.broadcast_to(obs_row, (d0, blk)).T
    mpos = obs == 2
    mneg = obs == 0
    d = x.shape[1]
    xb = x.astype(bf)
    h_raw = jax.lax.dot(xb, w1c[...], preferred_element_type=f32)
    b1 = b1c[...]
    hp = jnp.where(mpos, jnp.maximum(h_raw[:, :d] + b1[:, :d], 0.0), 0.0).astype(bf)
    hn = jnp.where(mneg, jnp.maximum(h_raw[:, d:] + b1[:, d:], 0.0), 0.0).astype(bf)
    h = jnp.concatenate([hp, hn], axis=1)
    y_raw = jax.lax.dot(h, w2c[...], preferred_element_type=f32)
    b2 = jnp.where(mpos, pb2[...], nb2[...])
    out_ref[...] = jnp.where(obs == 1, x, jnp.maximum(y_raw + b2, 0.0))

    @pl.when(pl.program_id(0) == 0)
    def _copy_edges():
        eout_ref[...] = e_ref[...]


def kernel(node_data, observations, edge_weights, pos_W1, pos_b1, pos_W2, pos_b2,
           neg_W1, neg_b1, neg_W2, neg_b2):
    n, d = node_data.shape
    e = edge_weights.shape[0]
    blk = _BLK
    g = n // blk
    obs = observations.astype(jnp.int32).reshape(g, 1, blk)
    bf = jnp.bfloat16
    w1c = jnp.concatenate([pos_W1.T, neg_W1.T], axis=1).astype(bf)
    b1c = jnp.concatenate([pos_b1, neg_b1]).reshape(1, 2 * d)
    w2c = jnp.concatenate([pos_W2.T, neg_W2.T], axis=0).astype(bf)
    full = lambda i: (0, 0)
    perm = lambda i: ((i * 3) % g, 0)
    out, eout = pl.pallas_call(
        _fused_block,
        grid=(g,),
        in_specs=[
            pl.BlockSpec((1, 1, blk), lambda i: ((i * 3) % g, 0, 0)),
            pl.BlockSpec((blk, d), perm),
            pl.BlockSpec((e,), lambda i: (0,)),
            pl.BlockSpec((d, 2 * d), full),
            pl.BlockSpec((1, 2 * d), full),
            pl.BlockSpec((2 * d, d), full),
            pl.BlockSpec((1, d), full),
            pl.BlockSpec((1, d), full),
        ],
        out_specs=[
            pl.BlockSpec((blk, d), perm),
            pl.BlockSpec((e,), lambda i: (0,)),
        ],
        out_shape=[
            jax.ShapeDtypeStruct((n, d), jnp.float32),
            jax.ShapeDtypeStruct((e,), jnp.float32),
        ],
        compiler_params=pltpu.CompilerParams(vmem_limit_bytes=114 * 1024 * 1024),
    )(
        obs, node_data, edge_weights,
        w1c, b1c, w2c, pos_b2.reshape(1, d), neg_b2.reshape(1, d),
    )
    return out, eout
